# baseline (speedup 1.0000x reference)
import functools

import jax
import jax.numpy as jnp
from jax import lax
from jax.experimental import pallas as pl
from jax.experimental.pallas import tpu as pltpu
from jax.experimental.pallas import tpu_sc as plsc

N = 10000
E = 320000
D = 128
L = 3

NC = 2
NS = 16
NW = NC * NS
NP = 10240
EPT = E // NW
CHUNK = 80
NCH = EPT // CHUNK
RPT = NP // NS


@functools.cache
def _sc_kernels():
    mesh = plsc.VectorSubcoreMesh(core_axis_name="c", subcore_axis_name="s",
                                  num_cores=NC, num_subcores=NS)
    params = pltpu.CompilerParams(needs_layout_passes=False)

    @functools.partial(
        pl.kernel,
        out_type=jax.ShapeDtypeStruct((NC, NP, D), jnp.float32),
        mesh=mesh,
        compiler_params=params,
        scratch_types=[
            pltpu.VMEM((CHUNK,), jnp.int32),
            pltpu.VMEM((CHUNK, D), jnp.float32),
            pltpu.VMEM_SHARED((NP, D), jnp.float32),
        ],
    )
    def deg_kernel(dst_hbm, out_hbm, dst_v, ones_v, acc):
        c = lax.axis_index("c")
        s = lax.axis_index("s")

        def _fill(val):
            def body(i, _):
                for g in range(D // 16):
                    ones_v[i, pl.ds(g * 16, 16)] = jnp.full((16,), val,
                                                            jnp.float32)
                return 0
            lax.fori_loop(0, CHUNK, body, 0)

        _fill(0.0)
        for k in range(RPT // CHUNK):
            pltpu.sync_copy(ones_v, acc.at[pl.ds(s * RPT + k * CHUNK, CHUNK)])
        _fill(1.0)
        plsc.subcore_barrier()

        base = (c * NS + s) * EPT

        def chunk_body(k, _):
            pltpu.sync_copy(dst_hbm.at[pl.ds(base + k * CHUNK, CHUNK)], dst_v)
            pltpu.sync_copy(ones_v, acc.at[dst_v], add=True)
            return 0

        lax.fori_loop(0, NCH, chunk_body, 0)
        plsc.subcore_barrier()
        pltpu.sync_copy(acc.at[pl.ds(s * RPT, RPT)],
                        out_hbm.at[c, pl.ds(s * RPT, RPT)])

    @functools.partial(
        pl.kernel,
        out_type=jax.ShapeDtypeStruct((NC, NP, D), jnp.float32),
        mesh=mesh,
        compiler_params=params,
        scratch_types=[
            pltpu.VMEM((CHUNK,), jnp.int32),
            pltpu.VMEM((CHUNK,), jnp.int32),
            pltpu.VMEM((CHUNK,), jnp.float32),
            pltpu.VMEM((CHUNK, D), jnp.float32),
            pltpu.VMEM_SHARED((NP, D), jnp.float32),
            pltpu.SemaphoreType.DMA,
        ],
    )
    def agg_kernel(x_hbm, src_hbm, dst_hbm, w_hbm, out_hbm,
                   src_v, dst_v, w_v, rows_v, acc, sem):
        c = lax.axis_index("c")
        s = lax.axis_index("s")

        def zbody(i, _):
            for g in range(D // 16):
                rows_v[i, pl.ds(g * 16, 16)] = jnp.zeros((16,), jnp.float32)
            return 0
        lax.fori_loop(0, CHUNK, zbody, 0)
        for k in range(RPT // CHUNK):
            pltpu.sync_copy(rows_v, acc.at[pl.ds(s * RPT + k * CHUNK, CHUNK)])
        plsc.subcore_barrier()

        base = (c * NS + s) * EPT
        iota = lax.iota(jnp.int32, 16)

        def chunk_body(k, _):
            off = base + k * CHUNK
            pltpu.sync_copy(src_hbm.at[pl.ds(off, CHUNK)], src_v)
            pltpu.sync_copy(w_hbm.at[pl.ds(off, CHUNK)], w_v)
            pltpu.sync_copy(dst_hbm.at[pl.ds(off, CHUNK)], dst_v)
            pltpu.async_copy(x_hbm.at[src_v], rows_v, sem).wait()

            def mul_body(t, _):
                wv = w_v[pl.ds(t * 16, 16)]
                rows16 = iota + t * 16
                for col in range(D):
                    cc = jnp.full((16,), col, jnp.int32)
                    v = plsc.load_gather(rows_v, [rows16, cc])
                    plsc.store_scatter(rows_v, [rows16, cc], v * wv)
                return 0
            lax.fori_loop(0, CHUNK // 16, mul_body, 0)

            pltpu.sync_copy(rows_v, acc.at[dst_v], add=True)
            return 0

        lax.fori_loop(0, NCH, chunk_body, 0)
        plsc.subcore_barrier()
        pltpu.sync_copy(acc.at[pl.ds(s * RPT, RPT)],
                        out_hbm.at[c, pl.ds(s * RPT, RPT)])

    return deg_kernel, agg_kernel


def _rdeg_body(degp_ref, out_ref):
    d = degp_ref[0] + degp_ref[1]
    out_ref[...] = 1.0 / jnp.maximum(d, 1.0)


BLK = 512
GRID = NP // BLK


def _rdeg_call(degp):
    return pl.pallas_call(
        _rdeg_body,
        grid=(GRID,),
        in_specs=[pl.BlockSpec((NC, BLK, D), lambda i: (0, i, 0))],
        out_specs=pl.BlockSpec((BLK, D), lambda i: (i, 0)),
        out_shape=jax.ShapeDtypeStruct((NP, D), jnp.float32),
    )(degp)


def _ln(h, g, b):
    mu = jnp.mean(h, axis=-1, keepdims=True)
    var = jnp.mean((h - mu) ** 2, axis=-1, keepdims=True)
    return (h - mu) / jnp.sqrt(var + 1e-5) * g + b


def _dot(a, b):
    return jnp.dot(a, b, precision=lax.Precision.HIGHEST,
                   preferred_element_type=jnp.float32)


def _layer_body_first(aggp, x, rdeg, wrelT, brel, wrootT, w1T, b1, g1, be1,
                      w2T, b2, g2, be2, wjkT, bjk, xout, jkout):
    agg = (aggp[0] + aggp[1]) * rdeg[...]
    x1 = _dot(agg, wrelT[...]) + brel[...] + _dot(x[...], wrootT[...])
    x2 = jax.nn.relu(_ln(_dot(x1, w1T[...]) + b1[...], g1[...], be1[...]))
    x3 = jax.nn.relu(_ln(_dot(x2, w2T[...]) + b2[...], g2[...], be2[...]))
    xout[...] = x3
    jkout[...] = bjk[...] + _dot(x3, wjkT[...])


def _layer_body_rest(aggp, x, rdeg, jk, wrelT, brel, wrootT, w1T, b1, g1, be1,
                     w2T, b2, g2, be2, wjkT, xout, jkout):
    agg = (aggp[0] + aggp[1]) * rdeg[...]
    x1 = _dot(agg, wrelT[...]) + brel[...] + _dot(x[...], wrootT[...])
    x2 = jax.nn.relu(_ln(_dot(x1, w1T[...]) + b1[...], g1[...], be1[...]))
    x3 = jax.nn.relu(_ln(_dot(x2, w2T[...]) + b2[...], g2[...], be2[...]))
    xout[...] = x3
    jkout[...] = jk[...] + _dot(x3, wjkT[...])


_ROWS = pl.BlockSpec((BLK, D), lambda i: (i, 0))
_AGGP = pl.BlockSpec((NC, BLK, D), lambda i: (0, i, 0))
_WMAT = pl.BlockSpec((D, D), lambda i: (0, 0))
_VEC = pl.BlockSpec((1, D), lambda i: (0, 0))


def _layer_call(aggp, x, rdeg, jk, wrelT, brel, wrootT, w1T, b1, g1, be1,
                w2T, b2, g2, be2, wjkT, bjk):
    out_shape = [jax.ShapeDtypeStruct((NP, D), jnp.float32),
                 jax.ShapeDtypeStruct((NP, D), jnp.float32)]
    wspecs = [_WMAT, _VEC, _WMAT, _WMAT, _VEC, _VEC, _VEC,
              _WMAT, _VEC, _VEC, _VEC, _WMAT]
    if jk is None:
        return pl.pallas_call(
            _layer_body_first,
            grid=(GRID,),
            in_specs=[_AGGP, _ROWS, _ROWS] + wspecs + [_VEC],
            out_specs=[_ROWS, _ROWS],
            out_shape=out_shape,
        )(aggp, x, rdeg, wrelT, brel, wrootT, w1T, b1, g1, be1,
          w2T, b2, g2, be2, wjkT, bjk)
    return pl.pallas_call(
        _layer_body_rest,
        grid=(GRID,),
        in_specs=[_AGGP, _ROWS, _ROWS, _ROWS] + wspecs,
        out_specs=[_ROWS, _ROWS],
        out_shape=out_shape,
    )(aggp, x, rdeg, jk, wrelT, brel, wrootT, w1T, b1, g1, be1,
      w2T, b2, g2, be2, wjkT)


def kernel(node, edge_index, edge_attr, batch_ptr, Wrel, brel, Wroot,
           W1, b1, W2, b2, g1, be1, g2, be2, Wjk, bjk):
    deg_kernel, agg_kernel = _sc_kernels()
    src = edge_index[0]
    dst = edge_index[1]

    xp = jnp.pad(node, ((0, NP - N), (0, 0)))

    degp = deg_kernel(dst)
    rdeg = _rdeg_call(degp)

    jk = None
    x = xp
    for i in range(L):
        aggp = agg_kernel(x, src, dst, edge_attr)
        wjkT = Wjk[:, i * D:(i + 1) * D].T
        x, jk = _layer_call(
            aggp, x, rdeg, jk,
            Wrel[i].T, brel[i].reshape(1, D), Wroot[i].T,
            W1[i].T, b1[i].reshape(1, D), g1[i].reshape(1, D),
            be1[i].reshape(1, D),
            W2[i].T, b2[i].reshape(1, D), g2[i].reshape(1, D),
            be2[i].reshape(1, D),
            wjkT, bjk.reshape(1, D))

    return jk[:N]


# speedup vs baseline: 6.8380x; 6.8380x over previous
import functools

import jax
import jax.numpy as jnp
from jax import lax
from jax.experimental import pallas as pl
from jax.experimental.pallas import tpu as pltpu
from jax.experimental.pallas import tpu_sc as plsc

N = 10000
E = 320000
D = 128
L = 3

NC = 2
NS = 16
NW = NC * NS
NP = 10240
EPT = E // NW
CHUNK = 80
NCH = EPT // CHUNK
RPT = NP // NS


@functools.cache
def _sc_kernels():
    mesh = plsc.VectorSubcoreMesh(core_axis_name="c", subcore_axis_name="s",
                                  num_cores=NC, num_subcores=NS)
    params = pltpu.CompilerParams(needs_layout_passes=False)

    @functools.partial(
        pl.kernel,
        out_type=jax.ShapeDtypeStruct((NC, NP, D), jnp.float32),
        mesh=mesh,
        compiler_params=params,
        scratch_types=[
            pltpu.VMEM((CHUNK,), jnp.int32),
            pltpu.VMEM((CHUNK, D), jnp.float32),
            pltpu.VMEM_SHARED((NP, D), jnp.float32),
        ],
    )
    def deg_kernel(dst_hbm, out_hbm, dst_v, ones_v, acc):
        c = lax.axis_index("c")
        s = lax.axis_index("s")

        def _fill(val):
            def body(i, _):
                for g in range(D // 16):
                    ones_v[i, pl.ds(g * 16, 16)] = jnp.full((16,), val,
                                                            jnp.float32)
                return 0
            lax.fori_loop(0, CHUNK, body, 0)

        _fill(0.0)
        for k in range(RPT // CHUNK):
            pltpu.sync_copy(ones_v, acc.at[pl.ds(s * RPT + k * CHUNK, CHUNK)])
        _fill(1.0)
        plsc.subcore_barrier()

        base = (c * NS + s) * EPT

        def chunk_body(k, _):
            pltpu.sync_copy(dst_hbm.at[pl.ds(base + k * CHUNK, CHUNK)], dst_v)
            pltpu.sync_copy(ones_v, acc.at[dst_v], add=True)
            return 0

        lax.fori_loop(0, NCH, chunk_body, 0)
        plsc.subcore_barrier()
        pltpu.sync_copy(acc.at[pl.ds(s * RPT, RPT)],
                        out_hbm.at[c, pl.ds(s * RPT, RPT)])

    @functools.partial(
        pl.kernel,
        out_type=jax.ShapeDtypeStruct((NC, NP, D), jnp.float32),
        mesh=mesh,
        compiler_params=params,
        scratch_types=[
            pltpu.VMEM((2, CHUNK), jnp.int32),
            pltpu.VMEM((2, CHUNK), jnp.int32),
            pltpu.VMEM((2, CHUNK), jnp.float32),
            pltpu.VMEM((CHUNK, D), jnp.float32),
            pltpu.VMEM((CHUNK, D), jnp.float32),
            pltpu.VMEM_SHARED((NP, D), jnp.float32),
            pltpu.SemaphoreType.DMA,
            pltpu.SemaphoreType.DMA,
            pltpu.SemaphoreType.DMA,
            pltpu.SemaphoreType.DMA,
        ],
    )
    def agg_kernel(x_hbm, src2_hbm, dst2_hbm, w2_hbm, out_hbm,
                   sv, dv, wv2, r0, r1, acc, i0, i1, s0, s1):
        c = lax.axis_index("c")
        s = lax.axis_index("s")
        rows_b = (r0, r1)
        isem = (i0, i1)
        rsem = (s0, s1)

        def zbody(i, _):
            for g in range(D // 16):
                r0[i, pl.ds(g * 16, 16)] = jnp.zeros((16,), jnp.float32)
            return 0
        lax.fori_loop(0, CHUNK, zbody, 0)
        for k in range(RPT // CHUNK):
            pltpu.sync_copy(r0, acc.at[pl.ds(s * RPT + k * CHUNK, CHUNK)])
        plsc.subcore_barrier()

        wid = c * NS + s

        def idx_start(chunk, p):
            pltpu.async_copy(src2_hbm.at[wid, chunk], sv.at[p], isem[p])
            pltpu.async_copy(dst2_hbm.at[wid, chunk], dv.at[p], isem[p])
            pltpu.async_copy(w2_hbm.at[wid, chunk], wv2.at[p], isem[p])

        def idx_wait(chunk, p):
            pltpu.make_async_copy(src2_hbm.at[wid, chunk], sv.at[p], isem[p]).wait()
            pltpu.make_async_copy(dst2_hbm.at[wid, chunk], dv.at[p], isem[p]).wait()
            pltpu.make_async_copy(w2_hbm.at[wid, chunk], wv2.at[p], isem[p]).wait()

        def row_start(p):
            pltpu.async_copy(x_hbm.at[sv.at[p]], rows_b[p], rsem[p])

        def row_wait(p):
            pltpu.make_async_copy(x_hbm.at[sv.at[p]], rows_b[p], rsem[p]).wait()

        def mul(p):
            rows = rows_b[p]

            def grp(t, _):
                wvv = wv2[p, pl.ds(t * 16, 16)]
                for e in range(16):
                    row = t * 16 + e
                    wb = wvv.at[jnp.full((16,), e, jnp.int32)].get(
                        mode="promise_in_bounds")
                    for g in range(D // 16):
                        v = rows[row, pl.ds(g * 16, 16)]
                        rows[row, pl.ds(g * 16, 16)] = v * wb
                return 0
            lax.fori_loop(0, CHUNK // 16, grp, 0)

        def scat(p):
            pltpu.sync_copy(rows_b[p], acc.at[dv.at[p]], add=True)

        idx_start(0, 0)
        idx_start(1, 1)
        idx_wait(0, 0)
        row_start(0)

        def pair(k2, _):
            a = 2 * k2
            b = a + 1
            idx_wait(b, 1)
            row_start(1)
            row_wait(0)
            mul(0)
            scat(0)
            idx_start(a + 2, 0)
            idx_wait(a + 2, 0)
            row_start(0)
            row_wait(1)
            mul(1)
            scat(1)

            @pl.when(b + 2 < NCH)
            def _():
                idx_start(b + 2, 1)
            return 0

        lax.fori_loop(0, (NCH - 1) // 2, pair, 0)
        row_wait(0)
        mul(0)
        scat(0)

        plsc.subcore_barrier()
        pltpu.sync_copy(acc.at[pl.ds(s * RPT, RPT)],
                        out_hbm.at[c, pl.ds(s * RPT, RPT)])

    return deg_kernel, agg_kernel


def _rdeg_body(degp_ref, out_ref):
    d = degp_ref[0] + degp_ref[1]
    out_ref[...] = 1.0 / jnp.maximum(d, 1.0)


BLK = 512
GRID = NP // BLK


def _rdeg_call(degp):
    return pl.pallas_call(
        _rdeg_body,
        grid=(GRID,),
        in_specs=[pl.BlockSpec((NC, BLK, D), lambda i: (0, i, 0))],
        out_specs=pl.BlockSpec((BLK, D), lambda i: (i, 0)),
        out_shape=jax.ShapeDtypeStruct((NP, D), jnp.float32),
    )(degp)


def _ln(h, g, b):
    mu = jnp.mean(h, axis=-1, keepdims=True)
    var = jnp.mean((h - mu) ** 2, axis=-1, keepdims=True)
    return (h - mu) / jnp.sqrt(var + 1e-5) * g + b


def _dot(a, b):
    return jnp.dot(a, b, precision=lax.Precision.HIGHEST,
                   preferred_element_type=jnp.float32)


def _layer_body_first(aggp, x, rdeg, wrelT, brel, wrootT, w1T, b1, g1, be1,
                      w2T, b2, g2, be2, wjkT, bjk, xout, jkout):
    agg = (aggp[0] + aggp[1]) * rdeg[...]
    x1 = _dot(agg, wrelT[...]) + brel[...] + _dot(x[...], wrootT[...])
    x2 = jax.nn.relu(_ln(_dot(x1, w1T[...]) + b1[...], g1[...], be1[...]))
    x3 = jax.nn.relu(_ln(_dot(x2, w2T[...]) + b2[...], g2[...], be2[...]))
    xout[...] = x3
    jkout[...] = bjk[...] + _dot(x3, wjkT[...])


def _layer_body_rest(aggp, x, rdeg, jk, wrelT, brel, wrootT, w1T, b1, g1, be1,
                     w2T, b2, g2, be2, wjkT, xout, jkout):
    agg = (aggp[0] + aggp[1]) * rdeg[...]
    x1 = _dot(agg, wrelT[...]) + brel[...] + _dot(x[...], wrootT[...])
    x2 = jax.nn.relu(_ln(_dot(x1, w1T[...]) + b1[...], g1[...], be1[...]))
    x3 = jax.nn.relu(_ln(_dot(x2, w2T[...]) + b2[...], g2[...], be2[...]))
    xout[...] = x3
    jkout[...] = jk[...] + _dot(x3, wjkT[...])


_ROWS = pl.BlockSpec((BLK, D), lambda i: (i, 0))
_AGGP = pl.BlockSpec((NC, BLK, D), lambda i: (0, i, 0))
_WMAT = pl.BlockSpec((D, D), lambda i: (0, 0))
_VEC = pl.BlockSpec((1, D), lambda i: (0, 0))


def _layer_call(aggp, x, rdeg, jk, wrelT, brel, wrootT, w1T, b1, g1, be1,
                w2T, b2, g2, be2, wjkT, bjk):
    out_shape = [jax.ShapeDtypeStruct((NP, D), jnp.float32),
                 jax.ShapeDtypeStruct((NP, D), jnp.float32)]
    wspecs = [_WMAT, _VEC, _WMAT, _WMAT, _VEC, _VEC, _VEC,
              _WMAT, _VEC, _VEC, _VEC, _WMAT]
    if jk is None:
        return pl.pallas_call(
            _layer_body_first,
            grid=(GRID,),
            in_specs=[_AGGP, _ROWS, _ROWS] + wspecs + [_VEC],
            out_specs=[_ROWS, _ROWS],
            out_shape=out_shape,
        )(aggp, x, rdeg, wrelT, brel, wrootT, w1T, b1, g1, be1,
          w2T, b2, g2, be2, wjkT, bjk)
    return pl.pallas_call(
        _layer_body_rest,
        grid=(GRID,),
        in_specs=[_AGGP, _ROWS, _ROWS, _ROWS] + wspecs,
        out_specs=[_ROWS, _ROWS],
        out_shape=out_shape,
    )(aggp, x, rdeg, jk, wrelT, brel, wrootT, w1T, b1, g1, be1,
      w2T, b2, g2, be2, wjkT)


def kernel(node, edge_index, edge_attr, batch_ptr, Wrel, brel, Wroot,
           W1, b1, W2, b2, g1, be1, g2, be2, Wjk, bjk):
    deg_kernel, agg_kernel = _sc_kernels()
    src = edge_index[0]
    dst = edge_index[1]

    xp = jnp.pad(node, ((0, NP - N), (0, 0)))

    degp = deg_kernel(dst)
    rdeg = _rdeg_call(degp)

    src2 = src.reshape(NW, NCH, CHUNK)
    dst2 = dst.reshape(NW, NCH, CHUNK)
    w2 = edge_attr.reshape(NW, NCH, CHUNK)

    jk = None
    x = xp
    for i in range(L):
        aggp = agg_kernel(x, src2, dst2, w2)
        wjkT = Wjk[:, i * D:(i + 1) * D].T
        x, jk = _layer_call(
            aggp, x, rdeg, jk,
            Wrel[i].T, brel[i].reshape(1, D), Wroot[i].T,
            W1[i].T, b1[i].reshape(1, D), g1[i].reshape(1, D),
            be1[i].reshape(1, D),
            W2[i].T, b2[i].reshape(1, D), g2[i].reshape(1, D),
            be2[i].reshape(1, D),
            wjkT, bjk.reshape(1, D))

    return jk[:N]


# speedup vs baseline: 6.8429x; 1.0007x over previous
import functools

import jax
import jax.numpy as jnp
from jax import lax
from jax.experimental import pallas as pl
from jax.experimental.pallas import tpu as pltpu
from jax.experimental.pallas import tpu_sc as plsc

N = 10000
E = 320000
D = 128
L = 3

NC = 2
NS = 16
NW = NC * NS
NP = 10240
EPT = E // NW
CHUNK = 80
NCH = EPT // CHUNK
RPT = NP // NS


@functools.cache
def _sc_kernels():
    mesh = plsc.VectorSubcoreMesh(core_axis_name="c", subcore_axis_name="s",
                                  num_cores=NC, num_subcores=NS)
    params = pltpu.CompilerParams(needs_layout_passes=False)

    @functools.partial(
        pl.kernel,
        out_type=jax.ShapeDtypeStruct((NC, NP, D), jnp.float32),
        mesh=mesh,
        compiler_params=params,
        scratch_types=[
            pltpu.VMEM((CHUNK,), jnp.int32),
            pltpu.VMEM((CHUNK, D), jnp.float32),
            pltpu.VMEM_SHARED((NP, D), jnp.float32),
        ],
    )
    def deg_kernel(dst_hbm, out_hbm, dst_v, ones_v, acc):
        c = lax.axis_index("c")
        s = lax.axis_index("s")

        def _fill(val):
            def body(i, _):
                for g in range(D // 16):
                    ones_v[i, pl.ds(g * 16, 16)] = jnp.full((16,), val,
                                                            jnp.float32)
                return 0
            lax.fori_loop(0, CHUNK, body, 0)

        _fill(0.0)
        for k in range(RPT // CHUNK):
            pltpu.sync_copy(ones_v, acc.at[pl.ds(s * RPT + k * CHUNK, CHUNK)])
        _fill(1.0)
        plsc.subcore_barrier()

        base = (c * NS + s) * EPT

        def chunk_body(k, _):
            pltpu.sync_copy(dst_hbm.at[pl.ds(base + k * CHUNK, CHUNK)], dst_v)
            pltpu.sync_copy(ones_v, acc.at[dst_v], add=True)
            return 0

        lax.fori_loop(0, NCH, chunk_body, 0)
        plsc.subcore_barrier()
        pltpu.sync_copy(acc.at[pl.ds(s * RPT, RPT)],
                        out_hbm.at[c, pl.ds(s * RPT, RPT)])

    @functools.partial(
        pl.kernel,
        out_type=jax.ShapeDtypeStruct((NC, NP, D), jnp.float32),
        mesh=mesh,
        compiler_params=params,
        scratch_types=[
            pltpu.VMEM((4, CHUNK), jnp.int32),
            pltpu.VMEM((4, CHUNK), jnp.int32),
            pltpu.VMEM((4, CHUNK), jnp.float32),
            pltpu.VMEM((CHUNK, D), jnp.float32),
            pltpu.VMEM((CHUNK, D), jnp.float32),
            pltpu.VMEM((CHUNK, D), jnp.float32),
            pltpu.VMEM((CHUNK, D), jnp.float32),
            pltpu.SemaphoreType.DMA,
            pltpu.SemaphoreType.DMA,
            pltpu.SemaphoreType.DMA,
            pltpu.SemaphoreType.DMA,
            pltpu.SemaphoreType.DMA,
            pltpu.SemaphoreType.DMA,
            pltpu.SemaphoreType.DMA,
            pltpu.SemaphoreType.DMA,
            pltpu.SemaphoreType.DMA,
            pltpu.SemaphoreType.DMA,
            pltpu.SemaphoreType.DMA,
            pltpu.SemaphoreType.DMA,
            pltpu.VMEM_SHARED((NP, D), jnp.float32),
        ],
    )
    def agg_kernel(x_hbm, src2_hbm, dst2_hbm, w2_hbm, out_hbm,
                   sv, dv, wv2, ra, rb, rc, rd,
                   ia, ib, ic, id_, ga, gb, gc, gd, ta, tb, tc_, td, acc):
        c = lax.axis_index("c")
        s = lax.axis_index("s")
        rows_b = (ra, rb, rc, rd)
        isem = (ia, ib, ic, id_)
        gsem = (ga, gb, gc, gd)
        tsem = (ta, tb, tc_, td)

        def zbody(i, _):
            for g in range(D // 16):
                ra[i, pl.ds(g * 16, 16)] = jnp.zeros((16,), jnp.float32)
            return 0
        lax.fori_loop(0, CHUNK, zbody, 0)
        for k in range(RPT // CHUNK):
            pltpu.sync_copy(ra, acc.at[pl.ds(s * RPT + k * CHUNK, CHUNK)])
        plsc.subcore_barrier()

        wid = c * NS + s

        def idx_start(chunk, p):
            pltpu.async_copy(src2_hbm.at[wid, chunk], sv.at[p], isem[p])
            pltpu.async_copy(dst2_hbm.at[wid, chunk], dv.at[p], isem[p])
            pltpu.async_copy(w2_hbm.at[wid, chunk], wv2.at[p], isem[p])

        def idx_wait(chunk, p):
            pltpu.make_async_copy(src2_hbm.at[wid, chunk], sv.at[p], isem[p]).wait()
            pltpu.make_async_copy(dst2_hbm.at[wid, chunk], dv.at[p], isem[p]).wait()
            pltpu.make_async_copy(w2_hbm.at[wid, chunk], wv2.at[p], isem[p]).wait()

        def row_start(p):
            pltpu.async_copy(x_hbm.at[sv.at[p]], rows_b[p], gsem[p])

        def row_wait(p):
            pltpu.make_async_copy(x_hbm.at[sv.at[p]], rows_b[p], gsem[p]).wait()

        def mul(p):
            rows = rows_b[p]

            def grp(t, _):
                wvv = wv2[p, pl.ds(t * 16, 16)]
                for e in range(16):
                    row = t * 16 + e
                    wb = wvv.at[jnp.full((16,), e, jnp.int32)].get(
                        mode="promise_in_bounds")
                    for g in range(D // 16):
                        v = rows[row, pl.ds(g * 16, 16)]
                        rows[row, pl.ds(g * 16, 16)] = v * wb
                return 0
            lax.fori_loop(0, CHUNK // 16, grp, 0)

        def scat_start(p):
            pltpu.async_copy(rows_b[p], acc.at[dv.at[p]], tsem[p], add=True)

        def scat_wait(p):
            pltpu.make_async_copy(rows_b[p], acc.at[dv.at[p]], tsem[p]).wait()

        for p in range(4):
            idx_start(p, p)
        for p in range(4):
            idx_wait(p, p)
            row_start(p)

        def quad(k4, _):
            base4 = 4 * k4
            for p in range(4):
                ch = base4 + p
                row_wait(p)
                mul(p)
                scat_start(p)
                nxt = ch + 4

                @pl.when(nxt < NCH)
                def _():
                    scat_wait(p)
                    idx_start(nxt, p)
                    idx_wait(nxt, p)
                    row_start(p)

                @pl.when(nxt >= NCH)
                def _():
                    scat_wait(p)
            return 0

        lax.fori_loop(0, NCH // 4, quad, 0)
        row_wait(0)
        mul(0)
        scat_start(0)
        scat_wait(0)

        plsc.subcore_barrier()
        pltpu.sync_copy(acc.at[pl.ds(s * RPT, RPT)],
                        out_hbm.at[c, pl.ds(s * RPT, RPT)])

    return deg_kernel, agg_kernel


def _rdeg_body(degp_ref, out_ref):
    d = degp_ref[0] + degp_ref[1]
    out_ref[...] = 1.0 / jnp.maximum(d, 1.0)


BLK = 512
GRID = NP // BLK


def _rdeg_call(degp):
    return pl.pallas_call(
        _rdeg_body,
        grid=(GRID,),
        in_specs=[pl.BlockSpec((NC, BLK, D), lambda i: (0, i, 0))],
        out_specs=pl.BlockSpec((BLK, D), lambda i: (i, 0)),
        out_shape=jax.ShapeDtypeStruct((NP, D), jnp.float32),
    )(degp)


def _ln(h, g, b):
    mu = jnp.mean(h, axis=-1, keepdims=True)
    var = jnp.mean((h - mu) ** 2, axis=-1, keepdims=True)
    return (h - mu) / jnp.sqrt(var + 1e-5) * g + b


def _dot(a, b):
    return jnp.dot(a, b, precision=lax.Precision.HIGHEST,
                   preferred_element_type=jnp.float32)


def _layer_body_first(aggp, x, rdeg, wrelT, brel, wrootT, w1T, b1, g1, be1,
                      w2T, b2, g2, be2, wjkT, bjk, xout, jkout):
    agg = (aggp[0] + aggp[1]) * rdeg[...]
    x1 = _dot(agg, wrelT[...]) + brel[...] + _dot(x[...], wrootT[...])
    x2 = jax.nn.relu(_ln(_dot(x1, w1T[...]) + b1[...], g1[...], be1[...]))
    x3 = jax.nn.relu(_ln(_dot(x2, w2T[...]) + b2[...], g2[...], be2[...]))
    xout[...] = x3
    jkout[...] = bjk[...] + _dot(x3, wjkT[...])


def _layer_body_rest(aggp, x, rdeg, jk, wrelT, brel, wrootT, w1T, b1, g1, be1,
                     w2T, b2, g2, be2, wjkT, xout, jkout):
    agg = (aggp[0] + aggp[1]) * rdeg[...]
    x1 = _dot(agg, wrelT[...]) + brel[...] + _dot(x[...], wrootT[...])
    x2 = jax.nn.relu(_ln(_dot(x1, w1T[...]) + b1[...], g1[...], be1[...]))
    x3 = jax.nn.relu(_ln(_dot(x2, w2T[...]) + b2[...], g2[...], be2[...]))
    xout[...] = x3
    jkout[...] = jk[...] + _dot(x3, wjkT[...])


_ROWS = pl.BlockSpec((BLK, D), lambda i: (i, 0))
_AGGP = pl.BlockSpec((NC, BLK, D), lambda i: (0, i, 0))
_WMAT = pl.BlockSpec((D, D), lambda i: (0, 0))
_VEC = pl.BlockSpec((1, D), lambda i: (0, 0))


def _layer_call(aggp, x, rdeg, jk, wrelT, brel, wrootT, w1T, b1, g1, be1,
                w2T, b2, g2, be2, wjkT, bjk):
    out_shape = [jax.ShapeDtypeStruct((NP, D), jnp.float32),
                 jax.ShapeDtypeStruct((NP, D), jnp.float32)]
    wspecs = [_WMAT, _VEC, _WMAT, _WMAT, _VEC, _VEC, _VEC,
              _WMAT, _VEC, _VEC, _VEC, _WMAT]
    if jk is None:
        return pl.pallas_call(
            _layer_body_first,
            grid=(GRID,),
            in_specs=[_AGGP, _ROWS, _ROWS] + wspecs + [_VEC],
            out_specs=[_ROWS, _ROWS],
            out_shape=out_shape,
        )(aggp, x, rdeg, wrelT, brel, wrootT, w1T, b1, g1, be1,
          w2T, b2, g2, be2, wjkT, bjk)
    return pl.pallas_call(
        _layer_body_rest,
        grid=(GRID,),
        in_specs=[_AGGP, _ROWS, _ROWS, _ROWS] + wspecs,
        out_specs=[_ROWS, _ROWS],
        out_shape=out_shape,
    )(aggp, x, rdeg, jk, wrelT, brel, wrootT, w1T, b1, g1, be1,
      w2T, b2, g2, be2, wjkT)


def kernel(node, edge_index, edge_attr, batch_ptr, Wrel, brel, Wroot,
           W1, b1, W2, b2, g1, be1, g2, be2, Wjk, bjk):
    deg_kernel, agg_kernel = _sc_kernels()
    src = edge_index[0]
    dst = edge_index[1]

    xp = jnp.pad(node, ((0, NP - N), (0, 0)))

    degp = deg_kernel(dst)
    rdeg = _rdeg_call(degp)

    src2 = src.reshape(NW, NCH, CHUNK)
    dst2 = dst.reshape(NW, NCH, CHUNK)
    w2 = edge_attr.reshape(NW, NCH, CHUNK)

    jk = None
    x = xp
    for i in range(L):
        aggp = agg_kernel(x, src2, dst2, w2)
        wjkT = Wjk[:, i * D:(i + 1) * D].T
        x, jk = _layer_call(
            aggp, x, rdeg, jk,
            Wrel[i].T, brel[i].reshape(1, D), Wroot[i].T,
            W1[i].T, b1[i].reshape(1, D), g1[i].reshape(1, D),
            be1[i].reshape(1, D),
            W2[i].T, b2[i].reshape(1, D), g2[i].reshape(1, D),
            be2[i].reshape(1, D),
            wjkT, bjk.reshape(1, D))

    return jk[:N]


# speedup vs baseline: 8.6243x; 1.2603x over previous
import functools

import jax
import jax.numpy as jnp
from jax import lax
from jax.experimental import pallas as pl
from jax.experimental.pallas import tpu as pltpu
from jax.experimental.pallas import tpu_sc as plsc

N = 10000
E = 320000
D = 128
L = 3

NC = 2
NS = 16
NW = NC * NS
NP = 10240
EPT = E // NW
CHUNK = 80
NCH = EPT // CHUNK
RPT = NP // NS


@functools.cache
def _sc_kernels():
    mesh = plsc.VectorSubcoreMesh(core_axis_name="c", subcore_axis_name="s",
                                  num_cores=NC, num_subcores=NS)
    params = pltpu.CompilerParams(needs_layout_passes=False)

    @functools.partial(
        pl.kernel,
        out_type=jax.ShapeDtypeStruct((NC, NP, D), jnp.float32),
        mesh=mesh,
        compiler_params=params,
        scratch_types=[
            pltpu.VMEM((CHUNK,), jnp.int32),
            pltpu.VMEM((CHUNK, D), jnp.float32),
            pltpu.VMEM_SHARED((NP, D), jnp.float32),
        ],
    )
    def deg_kernel(dst_hbm, out_hbm, dst_v, ones_v, acc):
        c = lax.axis_index("c")
        s = lax.axis_index("s")

        def _fill(val):
            def body(i, _):
                for g in range(D // 16):
                    ones_v[i, pl.ds(g * 16, 16)] = jnp.full((16,), val,
                                                            jnp.float32)
                return 0
            lax.fori_loop(0, CHUNK, body, 0)

        _fill(0.0)
        for k in range(RPT // CHUNK):
            pltpu.sync_copy(ones_v, acc.at[pl.ds(s * RPT + k * CHUNK, CHUNK)])
        _fill(1.0)
        plsc.subcore_barrier()

        base = (c * NS + s) * EPT

        def chunk_body(k, _):
            pltpu.sync_copy(dst_hbm.at[pl.ds(base + k * CHUNK, CHUNK)], dst_v)
            pltpu.sync_copy(ones_v, acc.at[dst_v], add=True)
            return 0

        lax.fori_loop(0, NCH, chunk_body, 0)
        plsc.subcore_barrier()
        pltpu.sync_copy(acc.at[pl.ds(s * RPT, RPT)],
                        out_hbm.at[c, pl.ds(s * RPT, RPT)])

    @functools.partial(
        pl.kernel,
        out_type=jax.ShapeDtypeStruct((NC, NP, D), jnp.float32),
        mesh=mesh,
        compiler_params=params,
        scratch_types=[
            pltpu.VMEM((8, CHUNK), jnp.int32),
            pltpu.VMEM((8, CHUNK), jnp.int32),
            pltpu.VMEM((8, 1, CHUNK), jnp.float32),
            pltpu.VMEM((CHUNK, D), jnp.float32),
            pltpu.VMEM((CHUNK, D), jnp.float32),
            pltpu.VMEM((CHUNK, D), jnp.float32),
            pltpu.VMEM((CHUNK, D), jnp.float32),
            [pltpu.SemaphoreType.DMA] * 8,
            [pltpu.SemaphoreType.DMA] * 4,
            [pltpu.SemaphoreType.DMA] * 4,
            pltpu.VMEM_SHARED((NP, D), jnp.float32),
        ],
    )
    def agg_kernel(x_hbm, src2_hbm, dst2_hbm, w2_hbm, out_hbm,
                   sv, dv, wv2, ra, rb, rc, rd, isem, gsem, tsem, acc):
        c = lax.axis_index("c")
        s = lax.axis_index("s")
        rows_b = (ra, rb, rc, rd)

        def zbody(i, _):
            for g in range(D // 16):
                ra[i, pl.ds(g * 16, 16)] = jnp.zeros((16,), jnp.float32)
            return 0
        lax.fori_loop(0, CHUNK, zbody, 0)
        for k in range(RPT // CHUNK):
            pltpu.sync_copy(ra, acc.at[pl.ds(s * RPT + k * CHUNK, CHUNK)])
        plsc.subcore_barrier()

        wid = c * NS + s

        def idx_start(chunk, p):
            pltpu.async_copy(src2_hbm.at[wid, pl.ds(chunk, 1)],
                             sv.at[pl.ds(p, 1)], isem[p])
            pltpu.async_copy(dst2_hbm.at[wid, pl.ds(chunk, 1)],
                             dv.at[pl.ds(p, 1)], isem[p])
            pltpu.async_copy(w2_hbm.at[wid, pl.ds(chunk, 1)],
                             wv2.at[p], isem[p])

        def idx_wait(chunk, p):
            pltpu.make_async_copy(src2_hbm.at[wid, pl.ds(chunk, 1)],
                                  sv.at[pl.ds(p, 1)], isem[p]).wait()
            pltpu.make_async_copy(dst2_hbm.at[wid, pl.ds(chunk, 1)],
                                  dv.at[pl.ds(p, 1)], isem[p]).wait()
            pltpu.make_async_copy(w2_hbm.at[wid, pl.ds(chunk, 1)],
                                  wv2.at[p], isem[p]).wait()

        def row_start(ip, rp):
            pltpu.async_copy(x_hbm.at[sv.at[ip]], rows_b[rp], gsem[rp])

        def row_wait(ip, rp):
            pltpu.make_async_copy(x_hbm.at[sv.at[ip]], rows_b[rp], gsem[rp]).wait()

        def mul(ip, rp):
            rows = rows_b[rp]

            def grp(t, _):
                wvv = wv2[ip, 0, pl.ds(t * 16, 16)]
                for e in range(16):
                    row = t * 16 + e
                    wb = wvv.at[jnp.full((16,), e, jnp.int32)].get(
                        mode="promise_in_bounds")
                    for g in range(D // 16):
                        v = rows[row, pl.ds(g * 16, 16)]
                        rows[row, pl.ds(g * 16, 16)] = v * wb
                return 0
            lax.fori_loop(0, CHUNK // 16, grp, 0)

        def scat_start(ip, rp):
            pltpu.async_copy(rows_b[rp], acc.at[dv.at[ip]], tsem[rp], add=True)

        def scat_wait(ip, rp):
            pltpu.make_async_copy(rows_b[rp], acc.at[dv.at[ip]], tsem[rp]).wait()

        for k in range(6):
            idx_start(k, k)
        idx_wait(0, 0)
        row_start(0, 0)
        idx_wait(1, 1)
        row_start(1, 1)

        def oct_(k8, _):
            base8 = 8 * k8
            for j in range(8):
                ch = base8 + j
                ip = j
                rp = j % 4
                row_wait(ip, rp)
                mul(ip, rp)
                scat_start(ip, rp)

                @pl.when(ch >= 2)
                def _():
                    scat_wait((j - 2) % 8, (j - 2) % 4)
                nxt6 = ch + 6
                ip6 = (j + 6) % 8

                @pl.when(nxt6 < NCH)
                def _():
                    idx_start(nxt6, ip6)
                nxt2 = ch + 2
                ip2 = (j + 2) % 8
                rp2 = (j + 2) % 4

                @pl.when(nxt2 < NCH)
                def _():
                    idx_wait(nxt2, ip2)
                    row_start(ip2, rp2)
            return 0

        lax.fori_loop(0, NCH // 8, oct_, 0)

        for j2 in range(NCH - 8 * (NCH // 8)):
            ch = 8 * (NCH // 8) + j2
            ip = ch % 8
            rp = ch % 4
            row_wait(ip, rp)
            mul(ip, rp)
            scat_start(ip, rp)
            scat_wait((ch - 2) % 8, (ch - 2) % 4)
            nxt2 = ch + 2
            if nxt2 < NCH:
                idx_wait(nxt2, nxt2 % 8)
                row_start(nxt2 % 8, nxt2 % 4)
        scat_wait((NCH - 2) % 8, (NCH - 2) % 4)
        scat_wait((NCH - 1) % 8, (NCH - 1) % 4)

        plsc.subcore_barrier()
        pltpu.sync_copy(acc.at[pl.ds(s * RPT, RPT)],
                        out_hbm.at[c, pl.ds(s * RPT, RPT)])

    return deg_kernel, agg_kernel


def _rdeg_body(degp_ref, out_ref):
    d = degp_ref[0] + degp_ref[1]
    out_ref[...] = 1.0 / jnp.maximum(d, 1.0)


BLK = 512
GRID = NP // BLK


def _rdeg_call(degp):
    return pl.pallas_call(
        _rdeg_body,
        grid=(GRID,),
        in_specs=[pl.BlockSpec((NC, BLK, D), lambda i: (0, i, 0))],
        out_specs=pl.BlockSpec((BLK, D), lambda i: (i, 0)),
        out_shape=jax.ShapeDtypeStruct((NP, D), jnp.float32),
    )(degp)


def _ln(h, g, b):
    mu = jnp.mean(h, axis=-1, keepdims=True)
    var = jnp.mean((h - mu) ** 2, axis=-1, keepdims=True)
    return (h - mu) / jnp.sqrt(var + 1e-5) * g + b


def _dot(a, b):
    return jnp.dot(a, b, precision=lax.Precision.HIGHEST,
                   preferred_element_type=jnp.float32)


def _layer_body_first(aggp, x, rdeg, wrelT, brel, wrootT, w1T, b1, g1, be1,
                      w2T, b2, g2, be2, wjkT, bjk, xout, jkout):
    agg = (aggp[0] + aggp[1]) * rdeg[...]
    x1 = _dot(agg, wrelT[...]) + brel[...] + _dot(x[...], wrootT[...])
    x2 = jax.nn.relu(_ln(_dot(x1, w1T[...]) + b1[...], g1[...], be1[...]))
    x3 = jax.nn.relu(_ln(_dot(x2, w2T[...]) + b2[...], g2[...], be2[...]))
    xout[...] = x3
    jkout[...] = bjk[...] + _dot(x3, wjkT[...])


def _layer_body_rest(aggp, x, rdeg, jk, wrelT, brel, wrootT, w1T, b1, g1, be1,
                     w2T, b2, g2, be2, wjkT, xout, jkout):
    agg = (aggp[0] + aggp[1]) * rdeg[...]
    x1 = _dot(agg, wrelT[...]) + brel[...] + _dot(x[...], wrootT[...])
    x2 = jax.nn.relu(_ln(_dot(x1, w1T[...]) + b1[...], g1[...], be1[...]))
    x3 = jax.nn.relu(_ln(_dot(x2, w2T[...]) + b2[...], g2[...], be2[...]))
    xout[...] = x3
    jkout[...] = jk[...] + _dot(x3, wjkT[...])


_ROWS = pl.BlockSpec((BLK, D), lambda i: (i, 0))
_AGGP = pl.BlockSpec((NC, BLK, D), lambda i: (0, i, 0))
_WMAT = pl.BlockSpec((D, D), lambda i: (0, 0))
_VEC = pl.BlockSpec((1, D), lambda i: (0, 0))


def _layer_call(aggp, x, rdeg, jk, wrelT, brel, wrootT, w1T, b1, g1, be1,
                w2T, b2, g2, be2, wjkT, bjk):
    out_shape = [jax.ShapeDtypeStruct((NP, D), jnp.float32),
                 jax.ShapeDtypeStruct((NP, D), jnp.float32)]
    wspecs = [_WMAT, _VEC, _WMAT, _WMAT, _VEC, _VEC, _VEC,
              _WMAT, _VEC, _VEC, _VEC, _WMAT]
    if jk is None:
        return pl.pallas_call(
            _layer_body_first,
            grid=(GRID,),
            in_specs=[_AGGP, _ROWS, _ROWS] + wspecs + [_VEC],
            out_specs=[_ROWS, _ROWS],
            out_shape=out_shape,
        )(aggp, x, rdeg, wrelT, brel, wrootT, w1T, b1, g1, be1,
          w2T, b2, g2, be2, wjkT, bjk)
    return pl.pallas_call(
        _layer_body_rest,
        grid=(GRID,),
        in_specs=[_AGGP, _ROWS, _ROWS, _ROWS] + wspecs,
        out_specs=[_ROWS, _ROWS],
        out_shape=out_shape,
    )(aggp, x, rdeg, jk, wrelT, brel, wrootT, w1T, b1, g1, be1,
      w2T, b2, g2, be2, wjkT)


def kernel(node, edge_index, edge_attr, batch_ptr, Wrel, brel, Wroot,
           W1, b1, W2, b2, g1, be1, g2, be2, Wjk, bjk):
    deg_kernel, agg_kernel = _sc_kernels()
    src = edge_index[0]
    dst = edge_index[1]

    xp = jnp.pad(node, ((0, NP - N), (0, 0)))

    degp = deg_kernel(dst)
    rdeg = _rdeg_call(degp)

    src2 = src.reshape(NW, NCH, CHUNK)
    dst2 = dst.reshape(NW, NCH, CHUNK)
    w2 = edge_attr.reshape(NW, NCH, CHUNK)

    jk = None
    x = xp
    for i in range(L):
        aggp = agg_kernel(x, src2, dst2, w2)
        wjkT = Wjk[:, i * D:(i + 1) * D].T
        x, jk = _layer_call(
            aggp, x, rdeg, jk,
            Wrel[i].T, brel[i].reshape(1, D), Wroot[i].T,
            W1[i].T, b1[i].reshape(1, D), g1[i].reshape(1, D),
            be1[i].reshape(1, D),
            W2[i].T, b2[i].reshape(1, D), g2[i].reshape(1, D),
            be2[i].reshape(1, D),
            wjkT, bjk.reshape(1, D))

    return jk[:N]


# speedup vs baseline: 9.1750x; 1.0639x over previous
import functools

import jax
import jax.numpy as jnp
from jax import lax
from jax.experimental import pallas as pl
from jax.experimental.pallas import tpu as pltpu
from jax.experimental.pallas import tpu_sc as plsc

N = 10000
E = 320000
D = 128
L = 3

NC = 2
NS = 16
NW = NC * NS
NP = 10240
EPT = E // NW
CHUNK = 80
NCH = EPT // CHUNK
RPT = NP // NS


@functools.cache
def _sc_kernels():
    mesh = plsc.VectorSubcoreMesh(core_axis_name="c", subcore_axis_name="s",
                                  num_cores=NC, num_subcores=NS)
    params = pltpu.CompilerParams(needs_layout_passes=False)

    @functools.partial(
        pl.kernel,
        out_type=jax.ShapeDtypeStruct((NC, NP, D), jnp.float32),
        mesh=mesh,
        compiler_params=params,
        scratch_types=[
            pltpu.VMEM((8, CHUNK), jnp.int32),
            pltpu.VMEM((CHUNK, D), jnp.float32),
            [pltpu.SemaphoreType.DMA] * 8,
            [pltpu.SemaphoreType.DMA] * 8,
            pltpu.VMEM_SHARED((NP, D), jnp.float32),
        ],
    )
    def deg_kernel(dst2_hbm, out_hbm, dv, ones_v, isem, tsem, acc):
        c = lax.axis_index("c")
        s = lax.axis_index("s")

        def _fill(val):
            def body(i, _):
                for g in range(D // 16):
                    ones_v[i, pl.ds(g * 16, 16)] = jnp.full((16,), val,
                                                            jnp.float32)
                return 0
            lax.fori_loop(0, CHUNK, body, 0)

        _fill(0.0)
        for k in range(RPT // CHUNK):
            pltpu.sync_copy(ones_v, acc.at[pl.ds(s * RPT + k * CHUNK, CHUNK)])
        _fill(1.0)
        plsc.subcore_barrier()

        wid = c * NS + s

        def idx_start(chunk, p):
            pltpu.async_copy(dst2_hbm.at[wid, pl.ds(chunk, 1)],
                             dv.at[pl.ds(p, 1)], isem[p])

        def idx_wait(chunk, p):
            pltpu.make_async_copy(dst2_hbm.at[wid, pl.ds(chunk, 1)],
                                  dv.at[pl.ds(p, 1)], isem[p]).wait()

        def scat_start(p):
            pltpu.async_copy(ones_v, acc.at[dv.at[p]], tsem[p], add=True)

        def scat_wait(p):
            pltpu.make_async_copy(ones_v, acc.at[dv.at[p]], tsem[p]).wait()

        for k in range(6):
            idx_start(k, k)

        def oct_(k8, _):
            base8 = 8 * k8
            for j in range(8):
                ch = base8 + j
                idx_wait(ch, j)
                scat_start(j)

                @pl.when(ch >= 2)
                def _():
                    scat_wait((j - 2) % 8)
                nxt6 = ch + 6

                @pl.when(nxt6 < NCH)
                def _():
                    idx_start(nxt6, (j + 6) % 8)
            return 0

        lax.fori_loop(0, NCH // 8, oct_, 0)
        for j2 in range(NCH - 8 * (NCH // 8)):
            ch = 8 * (NCH // 8) + j2
            idx_wait(ch, ch % 8)
            scat_start(ch % 8)
            scat_wait((ch - 2) % 8)
        scat_wait((NCH - 2) % 8)
        scat_wait((NCH - 1) % 8)

        plsc.subcore_barrier()
        pltpu.sync_copy(acc.at[pl.ds(s * RPT, RPT)],
                        out_hbm.at[c, pl.ds(s * RPT, RPT)])

    @functools.partial(
        pl.kernel,
        out_type=jax.ShapeDtypeStruct((NC, NP, D), jnp.float32),
        mesh=mesh,
        compiler_params=params,
        scratch_types=[
            pltpu.VMEM((8, CHUNK), jnp.int32),
            pltpu.VMEM((8, CHUNK), jnp.int32),
            pltpu.VMEM((8, 1, CHUNK), jnp.float32),
            pltpu.VMEM((CHUNK, D), jnp.float32),
            pltpu.VMEM((CHUNK, D), jnp.float32),
            pltpu.VMEM((CHUNK, D), jnp.float32),
            pltpu.VMEM((CHUNK, D), jnp.float32),
            [pltpu.SemaphoreType.DMA] * 8,
            [pltpu.SemaphoreType.DMA] * 4,
            [pltpu.SemaphoreType.DMA] * 4,
            pltpu.VMEM_SHARED((NP, D), jnp.float32),
        ],
    )
    def agg_kernel(x_hbm, src2_hbm, dst2_hbm, w2_hbm, out_hbm,
                   sv, dv, wv2, ra, rb, rc, rd, isem, gsem, tsem, acc):
        c = lax.axis_index("c")
        s = lax.axis_index("s")
        rows_b = (ra, rb, rc, rd)

        def zbody(i, _):
            for g in range(D // 16):
                ra[i, pl.ds(g * 16, 16)] = jnp.zeros((16,), jnp.float32)
            return 0
        lax.fori_loop(0, CHUNK, zbody, 0)
        for k in range(RPT // CHUNK):
            pltpu.sync_copy(ra, acc.at[pl.ds(s * RPT + k * CHUNK, CHUNK)])
        plsc.subcore_barrier()

        wid = c * NS + s

        def idx_start(chunk, p):
            pltpu.async_copy(src2_hbm.at[wid, pl.ds(chunk, 1)],
                             sv.at[pl.ds(p, 1)], isem[p])
            pltpu.async_copy(dst2_hbm.at[wid, pl.ds(chunk, 1)],
                             dv.at[pl.ds(p, 1)], isem[p])
            pltpu.async_copy(w2_hbm.at[wid, pl.ds(chunk, 1)],
                             wv2.at[p], isem[p])

        def idx_wait(chunk, p):
            pltpu.make_async_copy(src2_hbm.at[wid, pl.ds(chunk, 1)],
                                  sv.at[pl.ds(p, 1)], isem[p]).wait()
            pltpu.make_async_copy(dst2_hbm.at[wid, pl.ds(chunk, 1)],
                                  dv.at[pl.ds(p, 1)], isem[p]).wait()
            pltpu.make_async_copy(w2_hbm.at[wid, pl.ds(chunk, 1)],
                                  wv2.at[p], isem[p]).wait()

        def row_start(ip, rp):
            pltpu.async_copy(x_hbm.at[sv.at[ip]], rows_b[rp], gsem[rp])

        def row_wait(ip, rp):
            pltpu.make_async_copy(x_hbm.at[sv.at[ip]], rows_b[rp], gsem[rp]).wait()

        def mul(ip, rp):
            rows = rows_b[rp]

            def grp(t, _):
                wvv = wv2[ip, 0, pl.ds(t * 16, 16)]
                for e in range(16):
                    row = t * 16 + e
                    wb = wvv.at[jnp.full((16,), e, jnp.int32)].get(
                        mode="promise_in_bounds")
                    for g in range(D // 16):
                        v = rows[row, pl.ds(g * 16, 16)]
                        rows[row, pl.ds(g * 16, 16)] = v * wb
                return 0
            lax.fori_loop(0, CHUNK // 16, grp, 0)

        def scat_start(ip, rp):
            pltpu.async_copy(rows_b[rp], acc.at[dv.at[ip]], tsem[rp], add=True)

        def scat_wait(ip, rp):
            pltpu.make_async_copy(rows_b[rp], acc.at[dv.at[ip]], tsem[rp]).wait()

        for k in range(6):
            idx_start(k, k)
        idx_wait(0, 0)
        row_start(0, 0)
        idx_wait(1, 1)
        row_start(1, 1)

        def oct_(k8, _):
            base8 = 8 * k8
            for j in range(8):
                ch = base8 + j
                ip = j
                rp = j % 4
                row_wait(ip, rp)
                mul(ip, rp)
                scat_start(ip, rp)

                @pl.when(ch >= 2)
                def _():
                    scat_wait((j - 2) % 8, (j - 2) % 4)
                nxt6 = ch + 6
                ip6 = (j + 6) % 8

                @pl.when(nxt6 < NCH)
                def _():
                    idx_start(nxt6, ip6)
                nxt2 = ch + 2
                ip2 = (j + 2) % 8
                rp2 = (j + 2) % 4

                @pl.when(nxt2 < NCH)
                def _():
                    idx_wait(nxt2, ip2)
                    row_start(ip2, rp2)
            return 0

        lax.fori_loop(0, NCH // 8, oct_, 0)

        for j2 in range(NCH - 8 * (NCH // 8)):
            ch = 8 * (NCH // 8) + j2
            ip = ch % 8
            rp = ch % 4
            row_wait(ip, rp)
            mul(ip, rp)
            scat_start(ip, rp)
            scat_wait((ch - 2) % 8, (ch - 2) % 4)
            nxt2 = ch + 2
            if nxt2 < NCH:
                idx_wait(nxt2, nxt2 % 8)
                row_start(nxt2 % 8, nxt2 % 4)
        scat_wait((NCH - 2) % 8, (NCH - 2) % 4)
        scat_wait((NCH - 1) % 8, (NCH - 1) % 4)

        plsc.subcore_barrier()
        pltpu.sync_copy(acc.at[pl.ds(s * RPT, RPT)],
                        out_hbm.at[c, pl.ds(s * RPT, RPT)])

    return deg_kernel, agg_kernel


BLK = 512
GRID = NP // BLK


def _ln(h, g, b):
    mu = jnp.mean(h, axis=-1, keepdims=True)
    var = jnp.mean((h - mu) ** 2, axis=-1, keepdims=True)
    return (h - mu) / jnp.sqrt(var + 1e-5) * g + b


def _dot(a, b):
    return jnp.dot(a, b, precision=lax.Precision.HIGHEST,
                   preferred_element_type=jnp.float32)


def _layer_body_first(degp, aggp, x, wrelT, brel, wrootT, w1T, b1, g1, be1,
                      w2T, b2, g2, be2, wjkT, bjk, xout, jkout, rdegout):
    rdeg = 1.0 / jnp.maximum(degp[0] + degp[1], 1.0)
    rdegout[...] = rdeg
    agg = (aggp[0] + aggp[1]) * rdeg
    x1 = _dot(agg, wrelT[...]) + brel[...] + _dot(x[...], wrootT[...])
    x2 = jax.nn.relu(_ln(_dot(x1, w1T[...]) + b1[...], g1[...], be1[...]))
    x3 = jax.nn.relu(_ln(_dot(x2, w2T[...]) + b2[...], g2[...], be2[...]))
    xout[...] = x3
    jkout[...] = bjk[...] + _dot(x3, wjkT[...])


def _layer_body_rest(aggp, x, rdeg, jk, wrelT, brel, wrootT, w1T, b1, g1, be1,
                     w2T, b2, g2, be2, wjkT, xout, jkout):
    agg = (aggp[0] + aggp[1]) * rdeg[...]
    x1 = _dot(agg, wrelT[...]) + brel[...] + _dot(x[...], wrootT[...])
    x2 = jax.nn.relu(_ln(_dot(x1, w1T[...]) + b1[...], g1[...], be1[...]))
    x3 = jax.nn.relu(_ln(_dot(x2, w2T[...]) + b2[...], g2[...], be2[...]))
    xout[...] = x3
    jkout[...] = jk[...] + _dot(x3, wjkT[...])


_ROWS = pl.BlockSpec((BLK, D), lambda i: (i, 0))
_AGGP = pl.BlockSpec((NC, BLK, D), lambda i: (0, i, 0))
_WMAT = pl.BlockSpec((D, D), lambda i: (0, 0))
_VEC = pl.BlockSpec((1, D), lambda i: (0, 0))


def _layer_call(aggp, x, rdeg, jk, wrelT, brel, wrootT, w1T, b1, g1, be1,
                w2T, b2, g2, be2, wjkT, bjk, degp=None):
    out_shape = [jax.ShapeDtypeStruct((NP, D), jnp.float32),
                 jax.ShapeDtypeStruct((NP, D), jnp.float32)]
    wspecs = [_WMAT, _VEC, _WMAT, _WMAT, _VEC, _VEC, _VEC,
              _WMAT, _VEC, _VEC, _VEC, _WMAT]
    if jk is None:
        return pl.pallas_call(
            _layer_body_first,
            grid=(GRID,),
            in_specs=[_AGGP, _AGGP, _ROWS] + wspecs + [_VEC],
            out_specs=[_ROWS, _ROWS, _ROWS],
            out_shape=out_shape + [jax.ShapeDtypeStruct((NP, D), jnp.float32)],
        )(degp, aggp, x, wrelT, brel, wrootT, w1T, b1, g1, be1,
          w2T, b2, g2, be2, wjkT, bjk)
    return pl.pallas_call(
        _layer_body_rest,
        grid=(GRID,),
        in_specs=[_AGGP, _ROWS, _ROWS, _ROWS] + wspecs,
        out_specs=[_ROWS, _ROWS],
        out_shape=out_shape,
    )(aggp, x, rdeg, jk, wrelT, brel, wrootT, w1T, b1, g1, be1,
      w2T, b2, g2, be2, wjkT)


def kernel(node, edge_index, edge_attr, batch_ptr, Wrel, brel, Wroot,
           W1, b1, W2, b2, g1, be1, g2, be2, Wjk, bjk):
    deg_kernel, agg_kernel = _sc_kernels()
    src = edge_index[0]
    dst = edge_index[1]

    xp = jnp.pad(node, ((0, NP - N), (0, 0)))

    src2 = src.reshape(NW, NCH, CHUNK)
    dst2 = dst.reshape(NW, NCH, CHUNK)
    w2 = edge_attr.reshape(NW, NCH, CHUNK)

    degp = deg_kernel(dst2)

    jk = None
    rdeg = None
    x = xp
    for i in range(L):
        aggp = agg_kernel(x, src2, dst2, w2)
        wjkT = Wjk[:, i * D:(i + 1) * D].T
        args = (x, rdeg, jk,
                Wrel[i].T, brel[i].reshape(1, D), Wroot[i].T,
                W1[i].T, b1[i].reshape(1, D), g1[i].reshape(1, D),
                be1[i].reshape(1, D),
                W2[i].T, b2[i].reshape(1, D), g2[i].reshape(1, D),
                be2[i].reshape(1, D),
                wjkT, bjk.reshape(1, D))
        if i == 0:
            x, jk, rdeg = _layer_call(aggp, *args, degp=degp)
        else:
            x, jk = _layer_call(aggp, *args)

    return jk[:N]
